# baseline (speedup 1.0000x reference)
import jax
import jax.numpy as jnp
from jax.experimental import pallas as pl
from jax.experimental.pallas import tpu as pltpu

_WINDOW = 20
_EPS = 1e-08


def _swen_kernel(mag_ref, bias_ref, norm_ref, mean_ref):
    m = mag_ref[0]
    F, T = m.shape
    fs = jnp.sum(m, axis=0, keepdims=True)
    win = fs
    for k in range(1, _WINDOW):
        shifted = jnp.concatenate(
            [jnp.zeros((1, k), fs.dtype), fs[:, : T - k]], axis=1)
        win = win + shifted
    t = jax.lax.broadcasted_iota(jnp.int32, (1, T), 1)
    count = jnp.minimum(t + 1, _WINDOW).astype(fs.dtype) * F
    mean = win / count + bias_ref[0, 0]
    mean_ref[0] = mean
    inv = 1.0 / (mean + _EPS)
    norm_ref[0] = m * inv


def kernel(mag, bias):
    B, F, T = mag.shape
    bias2 = bias.reshape(1, 1)
    norm, mean = pl.pallas_call(
        _swen_kernel,
        grid=(B,),
        in_specs=[
            pl.BlockSpec((1, F, T), lambda b: (b, 0, 0)),
            pl.BlockSpec((1, 1), lambda b: (0, 0)),
        ],
        out_specs=[
            pl.BlockSpec((1, F, T), lambda b: (b, 0, 0)),
            pl.BlockSpec((1, 1, T), lambda b: (b, 0, 0)),
        ],
        out_shape=[
            jax.ShapeDtypeStruct((B, F, T), mag.dtype),
            jax.ShapeDtypeStruct((B, 1, T), mag.dtype),
        ],
        compiler_params=pltpu.CompilerParams(
            dimension_semantics=("parallel",),
        ),
    )(mag, bias2)
    return norm, mean


# speedup vs baseline: 3.2869x; 3.2869x over previous
import jax
import jax.numpy as jnp
from jax.experimental import pallas as pl
from jax.experimental.pallas import tpu as pltpu

_WINDOW = 20
_EPS = 1e-08


def _swen_kernel(mag_ref, bias_ref, norm_ref, mean_ref, carry_ref):
    i = pl.program_id(0)
    m = mag_ref[...]
    F, B, TB = m.shape
    fs = jnp.sum(m, axis=0)

    @pl.when(i == 0)
    def _():
        carry_ref[...] = jnp.zeros_like(carry_ref)

    ext = jnp.concatenate([carry_ref[...], fs], axis=1)
    carry_ref[...] = ext[:, TB:]
    win = ext[:, 0:TB]
    for j in range(1, _WINDOW):
        win = win + ext[:, j:j + TB]

    t = i * TB + jax.lax.broadcasted_iota(jnp.int32, (B, TB), 1)
    count = jnp.minimum(t + 1, _WINDOW).astype(fs.dtype) * F
    mean = win / count + bias_ref[0, 0]
    mean_ref[:, 0, :] = mean
    inv = 1.0 / (mean + _EPS)
    norm_ref[...] = m * inv[None, :, :]


def kernel(mag, bias):
    B, F, T = mag.shape
    mag_t = jnp.transpose(mag, (1, 0, 2))
    TB = 512
    bias2 = bias.reshape(1, 1)
    norm_t, mean = pl.pallas_call(
        _swen_kernel,
        grid=(pl.cdiv(T, TB),),
        in_specs=[
            pl.BlockSpec((F, B, TB), lambda i: (0, 0, i)),
            pl.BlockSpec((1, 1), lambda i: (0, 0)),
        ],
        out_specs=[
            pl.BlockSpec((F, B, TB), lambda i: (0, 0, i)),
            pl.BlockSpec((B, 1, TB), lambda i: (0, 0, i)),
        ],
        out_shape=[
            jax.ShapeDtypeStruct((F, B, T), mag.dtype),
            jax.ShapeDtypeStruct((B, 1, T), mag.dtype),
        ],
        scratch_shapes=[pltpu.VMEM((B, _WINDOW - 1), jnp.float32)],
        compiler_params=pltpu.CompilerParams(
            dimension_semantics=("arbitrary",),
        ),
    )(mag_t, bias2)
    return jnp.transpose(norm_t, (1, 0, 2)), mean


# speedup vs baseline: 3.3749x; 1.0268x over previous
import jax
import jax.numpy as jnp
from jax.experimental import pallas as pl
from jax.experimental.pallas import tpu as pltpu

_WINDOW = 20
_EPS = 1e-08


def _swen_kernel(mag_ref, bias_ref, norm_ref, mean_ref, carry_ref):
    i = pl.program_id(0)
    m = mag_ref[...]
    F, B, TB = m.shape
    fs = jnp.sum(m, axis=0)

    @pl.when(i == 0)
    def _():
        carry_ref[...] = jnp.zeros_like(carry_ref)

    ext = jnp.concatenate([carry_ref[...], fs], axis=1)
    carry_ref[...] = ext[:, TB:]
    win = ext[:, 0:TB]
    for j in range(1, _WINDOW):
        win = win + ext[:, j:j + TB]

    t = i * TB + jax.lax.broadcasted_iota(jnp.int32, (B, TB), 1)
    count = jnp.minimum(t + 1, _WINDOW).astype(fs.dtype) * F
    mean = win / count + bias_ref[0, 0]
    mean_ref[:, 0, :] = mean
    inv = 1.0 / (mean + _EPS)
    norm_ref[...] = m * inv[None, :, :]


def kernel(mag, bias):
    B, F, T = mag.shape
    mag_t = jnp.transpose(mag, (1, 0, 2))
    TB = 640
    bias2 = bias.reshape(1, 1)
    norm_t, mean = pl.pallas_call(
        _swen_kernel,
        grid=(pl.cdiv(T, TB),),
        in_specs=[
            pl.BlockSpec((F, B, TB), lambda i: (0, 0, i)),
            pl.BlockSpec((1, 1), lambda i: (0, 0)),
        ],
        out_specs=[
            pl.BlockSpec((F, B, TB), lambda i: (0, 0, i)),
            pl.BlockSpec((B, 1, TB), lambda i: (0, 0, i)),
        ],
        out_shape=[
            jax.ShapeDtypeStruct((F, B, T), mag.dtype),
            jax.ShapeDtypeStruct((B, 1, T), mag.dtype),
        ],
        scratch_shapes=[pltpu.VMEM((B, _WINDOW - 1), jnp.float32)],
        compiler_params=pltpu.CompilerParams(
            dimension_semantics=("arbitrary",),
        ),
    )(mag_t, bias2)
    return jnp.transpose(norm_t, (1, 0, 2)), mean


# speedup vs baseline: 3.3788x; 1.0011x over previous
import jax
import jax.numpy as jnp
from jax.experimental import pallas as pl
from jax.experimental.pallas import tpu as pltpu

_WINDOW = 20
_EPS = 1e-08


def _swen_kernel(mag_ref, bias_ref, norm_ref, mean_ref, carry_ref):
    i = pl.program_id(0)
    m = mag_ref[...]
    F, B, TB = m.shape
    fs = jnp.sum(m, axis=0)

    @pl.when(i == 0)
    def _():
        carry_ref[...] = jnp.zeros_like(carry_ref)

    ext = jnp.concatenate([carry_ref[...], fs], axis=1)
    carry_ref[...] = ext[:, TB:]
    win = ext[:, 0:TB]
    for j in range(1, _WINDOW):
        win = win + ext[:, j:j + TB]

    t = i * TB + jax.lax.broadcasted_iota(jnp.int32, (B, TB), 1)
    count = jnp.minimum(t + 1, _WINDOW).astype(fs.dtype) * F
    mean = win / count + bias_ref[0]
    mean_ref[:, 0, :] = mean
    inv = 1.0 / (mean + _EPS)
    norm_ref[...] = m * inv[None, :, :]


def kernel(mag, bias):
    B, F, T = mag.shape
    mag_t = jnp.transpose(mag, (1, 0, 2))
    TB = 640
    norm_t, mean = pl.pallas_call(
        _swen_kernel,
        grid=(pl.cdiv(T, TB),),
        in_specs=[
            pl.BlockSpec((F, B, TB), lambda i: (0, 0, i)),
            pl.BlockSpec(memory_space=pltpu.SMEM),
        ],
        out_specs=[
            pl.BlockSpec((F, B, TB), lambda i: (0, 0, i)),
            pl.BlockSpec((B, 1, TB), lambda i: (0, 0, i)),
        ],
        out_shape=[
            jax.ShapeDtypeStruct((F, B, T), mag.dtype),
            jax.ShapeDtypeStruct((B, 1, T), mag.dtype),
        ],
        scratch_shapes=[pltpu.VMEM((B, _WINDOW - 1), jnp.float32)],
        compiler_params=pltpu.CompilerParams(
            dimension_semantics=("arbitrary",),
        ),
    )(mag_t, bias)
    return jnp.transpose(norm_t, (1, 0, 2)), mean


# speedup vs baseline: 3.4031x; 1.0072x over previous
import jax
import jax.numpy as jnp
from jax.experimental import pallas as pl
from jax.experimental.pallas import tpu as pltpu

_WINDOW = 20
_EPS = 1e-08


def _swen_kernel(mag_ref, bias_ref, norm_ref, mean_ref, carry_ref):
    i = pl.program_id(0)
    m = mag_ref[...]
    F, B, TB = m.shape
    fs = jnp.sum(m, axis=0)

    @pl.when(i == 0)
    def _():
        carry_ref[...] = jnp.zeros_like(carry_ref)

    ext = jnp.concatenate([carry_ref[...], fs], axis=1)
    carry_ref[...] = ext[:, TB:]
    win = ext[:, 0:TB]
    for j in range(1, _WINDOW):
        win = win + ext[:, j:j + TB]

    t = i * TB + jax.lax.broadcasted_iota(jnp.int32, (B, TB), 1)
    count = jnp.minimum(t + 1, _WINDOW).astype(fs.dtype) * F
    mean = win / count + bias_ref[0]
    mean_ref[:, 0, :] = mean
    inv = 1.0 / (mean + _EPS)
    norm_ref[...] = m * inv[None, :, :]


def kernel(mag, bias):
    B, F, T = mag.shape
    mag_t = jnp.transpose(mag, (1, 0, 2))
    TB = 768
    norm_t, mean = pl.pallas_call(
        _swen_kernel,
        grid=(pl.cdiv(T, TB),),
        in_specs=[
            pl.BlockSpec((F, B, TB), lambda i: (0, 0, i)),
            pl.BlockSpec(memory_space=pltpu.SMEM),
        ],
        out_specs=[
            pl.BlockSpec((F, B, TB), lambda i: (0, 0, i)),
            pl.BlockSpec((B, 1, TB), lambda i: (0, 0, i)),
        ],
        out_shape=[
            jax.ShapeDtypeStruct((F, B, T), mag.dtype),
            jax.ShapeDtypeStruct((B, 1, T), mag.dtype),
        ],
        scratch_shapes=[pltpu.VMEM((B, _WINDOW - 1), jnp.float32)],
        compiler_params=pltpu.CompilerParams(
            dimension_semantics=("arbitrary",),
            vmem_limit_bytes=100 * 1024 * 1024,
        ),
    )(mag_t, bias)
    return jnp.transpose(norm_t, (1, 0, 2)), mean
